# baseline (speedup 1.0000x reference)
import jax
import jax.numpy as jnp
import numpy as np
from jax.experimental import pallas as pl
from jax.experimental.pallas import tpu as pltpu

_BM = 512


def _proj_kernel(idx_ref, wt_ref, x_ref, out_ref, p_ref):
    @pl.when(pl.program_id(0) == 0)
    def _():
        K = wt_ref.shape[1]
        C = out_ref.shape[1]
        kio = jax.lax.broadcasted_iota(jnp.int32, (K, C), 0)
        s = jnp.zeros((K, C), jnp.float32)
        for j in range(idx_ref.shape[0]):
            s = s + (kio == idx_ref[j : j + 1, :]).astype(jnp.float32)
        p = jnp.dot(wt_ref[...], s, preferred_element_type=jnp.float32)
        inv_scale = np.float32(1.0 / np.sqrt(float(wt_ref.shape[0])))
        p_ref[...] = p * inv_scale

    out_ref[...] = jnp.dot(x_ref[...], p_ref[...],
                           preferred_element_type=jnp.float32)


def kernel(inputs, weight, feature_table):
    B, E = inputs.shape
    T = weight.shape[0]
    V, F = feature_table.shape
    C = V + 1

    wt = jnp.concatenate([weight, jnp.zeros((1, E), weight.dtype)], axis=0).T

    ftT = feature_table.T.astype(jnp.int32)
    pad_row = jnp.full((1, V), T, jnp.int32)
    ftT8 = jnp.concatenate([ftT, pad_row], axis=0)
    col0 = jnp.full((F + 1, 1), T, jnp.int32).at[0, 0].set(0)
    idx = jnp.concatenate([col0, ftT8], axis=1)

    grid = (B // _BM,)
    return pl.pallas_call(
        _proj_kernel,
        grid=grid,
        in_specs=[
            pl.BlockSpec((F + 1, C), lambda i: (0, 0)),
            pl.BlockSpec((E, T + 1), lambda i: (0, 0)),
            pl.BlockSpec((_BM, E), lambda i: (i, 0)),
        ],
        out_specs=pl.BlockSpec((_BM, C), lambda i: (i, 0)),
        out_shape=jax.ShapeDtypeStruct((B, C), jnp.float32),
        scratch_shapes=[pltpu.VMEM((E, C), jnp.float32)],
    )(idx, wt, inputs)


# speedup vs baseline: 1.5391x; 1.5391x over previous
import jax
import jax.numpy as jnp
import numpy as np
from jax.experimental import pallas as pl
from jax.experimental.pallas import tpu as pltpu

_BM = 4096


def _proj_kernel(idx_ref, wt_ref, x_ref, out_ref, p_ref):
    @pl.when(pl.program_id(0) == 0)
    def _():
        K = wt_ref.shape[1]
        C = out_ref.shape[1]
        kio = jax.lax.broadcasted_iota(jnp.int32, (K, C), 0)
        s = jnp.zeros((K, C), jnp.float32)
        for j in range(idx_ref.shape[0]):
            s = s + (kio == idx_ref[j : j + 1, :]).astype(jnp.float32)
        p = jnp.dot(wt_ref[...], s, preferred_element_type=jnp.float32)
        inv_scale = np.float32(1.0 / np.sqrt(float(wt_ref.shape[0])))
        p_ref[...] = p * inv_scale

    out_ref[...] = jnp.dot(x_ref[...], p_ref[...],
                           preferred_element_type=jnp.float32)


def kernel(inputs, weight, feature_table):
    B, E = inputs.shape
    T = weight.shape[0]
    V, F = feature_table.shape
    C = V + 1

    wt = jnp.concatenate([weight, jnp.zeros((1, E), weight.dtype)], axis=0).T

    ftT = feature_table.T.astype(jnp.int32)
    pad_row = jnp.full((1, V), T, jnp.int32)
    ftT8 = jnp.concatenate([ftT, pad_row], axis=0)
    col0 = jnp.full((F + 1, 1), T, jnp.int32).at[0, 0].set(0)
    idx = jnp.concatenate([col0, ftT8], axis=1)

    grid = (B // _BM,)
    return pl.pallas_call(
        _proj_kernel,
        grid=grid,
        in_specs=[
            pl.BlockSpec((F + 1, C), lambda i: (0, 0)),
            pl.BlockSpec((E, T + 1), lambda i: (0, 0)),
            pl.BlockSpec((_BM, E), lambda i: (i, 0)),
        ],
        out_specs=pl.BlockSpec((_BM, C), lambda i: (i, 0)),
        out_shape=jax.ShapeDtypeStruct((B, C), jnp.float32),
        scratch_shapes=[pltpu.VMEM((E, C), jnp.float32)],
    )(idx, wt, inputs)
